# baseline (speedup 1.0000x reference)
import functools

import jax
import jax.numpy as jnp
from jax import lax
from jax.experimental import pallas as pl
from jax.experimental.pallas import tpu as pltpu
from jax.experimental.pallas import tpu_sc as plsc

VOCAB = 1000000
EMB = 64
BATCH = 4096
SEQ = 200
TOK = BATCH * SEQ

NUM_CORES = 2
NUM_SUBCORES = 16
NW = NUM_CORES * NUM_SUBCORES
TOK_PER_W = TOK // NW

SUB = 128
K = 8
CHUNK = SUB * K
NCHUNK = TOK_PER_W // CHUNK

_mesh = plsc.VectorSubcoreMesh(
    core_axis_name="c", subcore_axis_name="s",
    num_cores=NUM_CORES, num_subcores=NUM_SUBCORES)


@functools.partial(
    pl.kernel,
    mesh=_mesh,
    compiler_params=pltpu.CompilerParams(use_tc_tiling_on_sc=False),
    out_type=jax.ShapeDtypeStruct((TOK, EMB), jnp.float32),
    scratch_types=[
        pltpu.VMEM((K, SUB), jnp.int32),
        pltpu.VMEM((CHUNK, EMB), jnp.float32),
        pltpu.SemaphoreType.DMA,
        pltpu.SemaphoreType.DMA,
    ],
)
def _gather_kernel(tok_hbm, table_hbm, out_hbm, idx_v, rows_v, sem_i, sem_g):
    wid = lax.axis_index("s") * NUM_CORES + lax.axis_index("c")
    base = wid * TOK_PER_W

    def body(g, carry):
        off = pl.multiple_of(base + g * CHUNK, CHUNK)
        row = pl.multiple_of(off // SUB, K)
        pltpu.async_copy(tok_hbm.at[pl.ds(row, K)], idx_v, sem_i).wait()
        copies = [
            pltpu.async_copy(
                table_hbm.at[idx_v.at[j]],
                rows_v.at[pl.ds(j * SUB, SUB)],
                sem_g,
            )
            for j in range(K)
        ]
        for c in copies:
            c.wait()
        pltpu.sync_copy(rows_v, out_hbm.at[pl.ds(off, CHUNK)])
        return carry

    lax.fori_loop(0, NCHUNK, body, 0)


def kernel(tokens, table):
    tok2d = tokens.reshape(TOK // SUB, SUB).astype(jnp.int32)
    out = _gather_kernel(tok2d, table)
    return out.reshape(BATCH, SEQ, EMB)


# speedup vs baseline: 1.0197x; 1.0197x over previous
import functools

import jax
import jax.numpy as jnp
from jax import lax
from jax.experimental import pallas as pl
from jax.experimental.pallas import tpu as pltpu
from jax.experimental.pallas import tpu_sc as plsc

VOCAB = 1000000
EMB = 64
BATCH = 4096
SEQ = 200
TOK = BATCH * SEQ

NUM_CORES = 2
NUM_SUBCORES = 16
NW = NUM_CORES * NUM_SUBCORES
TOK_PER_W = TOK // NW

SUB = 128
K = 5
CHUNK = SUB * K
NCHUNK = TOK_PER_W // CHUNK

_mesh = plsc.VectorSubcoreMesh(
    core_axis_name="c", subcore_axis_name="s",
    num_cores=NUM_CORES, num_subcores=NUM_SUBCORES)


@functools.partial(
    pl.kernel,
    mesh=_mesh,
    compiler_params=pltpu.CompilerParams(use_tc_tiling_on_sc=False),
    out_type=jax.ShapeDtypeStruct((TOK, EMB), jnp.float32),
    scratch_types=[
        pltpu.VMEM((CHUNK,), jnp.int32),
        pltpu.VMEM((CHUNK,), jnp.int32),
        pltpu.VMEM((CHUNK, EMB), jnp.float32),
        pltpu.VMEM((CHUNK, EMB), jnp.float32),
        pltpu.SemaphoreType.DMA,
        pltpu.SemaphoreType.DMA,
        pltpu.SemaphoreType.DMA,
        pltpu.SemaphoreType.DMA,
    ],
)
def _gather_kernel(tok_hbm, table_hbm, out_hbm,
                   idx_v0, idx_v1, rows_v0, rows_v1,
                   sem_g0, sem_g1, sem_o0, sem_o1):
    wid = lax.axis_index("s") * NUM_CORES + lax.axis_index("c")
    base = wid * TOK_PER_W
    idx_v = (idx_v0, idx_v1)
    rows_v = (rows_v0, rows_v1)
    sem_g = (sem_g0, sem_g1)
    sem_o = (sem_o0, sem_o1)

    def load_idx(g, b):
        off = pl.multiple_of(base + g * CHUNK, CHUNK)
        pltpu.sync_copy(tok_hbm.at[pl.ds(off, CHUNK)], idx_v[b])

    def fire_gathers(b):
        for j in range(K):
            pltpu.async_copy(
                table_hbm.at[idx_v[b].at[pl.ds(j * SUB, SUB)]],
                rows_v[b].at[pl.ds(j * SUB, SUB)],
                sem_g[b],
            )

    def wait_gathers(b):
        for j in range(K):
            pltpu.make_async_copy(
                table_hbm.at[idx_v[b].at[pl.ds(j * SUB, SUB)]],
                rows_v[b].at[pl.ds(j * SUB, SUB)],
                sem_g[b],
            ).wait()

    def fire_writeback(g, b):
        off = pl.multiple_of(base + g * CHUNK, CHUNK)
        pltpu.async_copy(rows_v[b], out_hbm.at[pl.ds(off, CHUNK)], sem_o[b])

    def wait_writeback(b):
        pltpu.make_async_copy(
            rows_v[b], out_hbm.at[pl.ds(base, CHUNK)], sem_o[b]).wait()

    load_idx(0, 0)
    fire_gathers(0)
    load_idx(1, 1)
    fire_gathers(1)
    wait_gathers(0)
    fire_writeback(0, 0)

    def body(s, carry):
        g0 = 2 * s
        load_idx(g0, 0)
        wait_writeback(0)
        fire_gathers(0)
        wait_gathers(1)
        fire_writeback(g0 - 1, 1)
        load_idx(g0 + 1, 1)
        wait_writeback(1)
        fire_gathers(1)
        wait_gathers(0)
        fire_writeback(g0, 0)
        return carry

    lax.fori_loop(1, NCHUNK // 2, body, 0)

    wait_gathers(1)
    fire_writeback(NCHUNK - 1, 1)
    wait_writeback(0)
    wait_writeback(1)


def kernel(tokens, table):
    tok_flat = tokens.reshape(TOK).astype(jnp.int32)
    out = _gather_kernel(tok_flat, table)
    return out.reshape(BATCH, SEQ, EMB)


# speedup vs baseline: 1.0214x; 1.0017x over previous
import functools

import jax
import jax.numpy as jnp
from jax import lax
from jax.experimental import pallas as pl
from jax.experimental.pallas import tpu as pltpu
from jax.experimental.pallas import tpu_sc as plsc

VOCAB = 1000000
EMB = 64
BATCH = 4096
SEQ = 200
TOK = BATCH * SEQ

NUM_CORES = 2
NUM_SUBCORES = 16
NW = NUM_CORES * NUM_SUBCORES
TOK_PER_W = TOK // NW

SUB = 640
K = 1
CHUNK = SUB * K
NCHUNK = TOK_PER_W // CHUNK

_mesh = plsc.VectorSubcoreMesh(
    core_axis_name="c", subcore_axis_name="s",
    num_cores=NUM_CORES, num_subcores=NUM_SUBCORES)


@functools.partial(
    pl.kernel,
    mesh=_mesh,
    compiler_params=pltpu.CompilerParams(use_tc_tiling_on_sc=False),
    out_type=jax.ShapeDtypeStruct((TOK, EMB), jnp.float32),
    scratch_types=[
        pltpu.VMEM((CHUNK,), jnp.int32),
        pltpu.VMEM((CHUNK,), jnp.int32),
        pltpu.VMEM((CHUNK, EMB), jnp.float32),
        pltpu.VMEM((CHUNK, EMB), jnp.float32),
        pltpu.SemaphoreType.DMA,
        pltpu.SemaphoreType.DMA,
        pltpu.SemaphoreType.DMA,
        pltpu.SemaphoreType.DMA,
    ],
)
def _gather_kernel(tok_hbm, table_hbm, out_hbm,
                   idx_v0, idx_v1, rows_v0, rows_v1,
                   sem_g0, sem_g1, sem_o0, sem_o1):
    wid = lax.axis_index("s") * NUM_CORES + lax.axis_index("c")
    base = wid * TOK_PER_W
    idx_v = (idx_v0, idx_v1)
    rows_v = (rows_v0, rows_v1)
    sem_g = (sem_g0, sem_g1)
    sem_o = (sem_o0, sem_o1)

    def load_idx(g, b):
        off = pl.multiple_of(base + g * CHUNK, CHUNK)
        pltpu.sync_copy(tok_hbm.at[pl.ds(off, CHUNK)], idx_v[b])

    def fire_gathers(b):
        for j in range(K):
            pltpu.async_copy(
                table_hbm.at[idx_v[b].at[pl.ds(j * SUB, SUB)]],
                rows_v[b].at[pl.ds(j * SUB, SUB)],
                sem_g[b],
            )

    def wait_gathers(b):
        for j in range(K):
            pltpu.make_async_copy(
                table_hbm.at[idx_v[b].at[pl.ds(j * SUB, SUB)]],
                rows_v[b].at[pl.ds(j * SUB, SUB)],
                sem_g[b],
            ).wait()

    def fire_writeback(g, b):
        off = pl.multiple_of(base + g * CHUNK, CHUNK)
        pltpu.async_copy(rows_v[b], out_hbm.at[pl.ds(off, CHUNK)], sem_o[b])

    def wait_writeback(b):
        pltpu.make_async_copy(
            rows_v[b], out_hbm.at[pl.ds(base, CHUNK)], sem_o[b]).wait()

    load_idx(0, 0)
    fire_gathers(0)
    load_idx(1, 1)
    fire_gathers(1)
    wait_gathers(0)
    fire_writeback(0, 0)

    def body(s, carry):
        g0 = 2 * s
        load_idx(g0, 0)
        wait_writeback(0)
        fire_gathers(0)
        wait_gathers(1)
        fire_writeback(g0 - 1, 1)
        load_idx(g0 + 1, 1)
        wait_writeback(1)
        fire_gathers(1)
        wait_gathers(0)
        fire_writeback(g0, 0)
        return carry

    lax.fori_loop(1, NCHUNK // 2, body, 0)

    wait_gathers(1)
    fire_writeback(NCHUNK - 1, 1)
    wait_writeback(0)
    wait_writeback(1)


def kernel(tokens, table):
    tok_flat = tokens.reshape(TOK).astype(jnp.int32)
    out = _gather_kernel(tok_flat, table)
    return out.reshape(BATCH, SEQ, EMB)
